# Initial kernel scaffold; baseline (speedup 1.0000x reference)
import jax
import jax.numpy as jnp
from jax.experimental import pallas as pl


def kernel(logits):
    raise NotImplementedError("write your pallas kernel here")


# speedup vs baseline: 6.5102x; 6.5102x over previous
import jax
import jax.numpy as jnp
from jax import lax
from jax.experimental import pallas as pl
from jax.experimental.pallas import tpu as pltpu
from jax.experimental.pallas import tpu_sc as plsc

OBS = 128
DIMS = 8192
LANES = 16
CHUNKS = DIMS // LANES
NC = 2
NS = 16
NW = NC * NS
RPW = OBS // NW
BISECT = 30


def _zeros():
    return jnp.zeros((LANES,), jnp.float32)


def _sparsemax_body(x_hbm, out_hbm, buf, cand):
    wid = lax.axis_index("s") * NC + lax.axis_index("c")
    base = wid * RPW
    pltpu.sync_copy(x_hbm.at[pl.ds(base, RPW)], buf)

    for r in range(RPW):
        def mx_body(i, acc):
            return jnp.maximum(acc, buf[r, pl.ds(i * LANES, LANES)])

        acc = lax.fori_loop(0, CHUNKS, mx_body,
                            jnp.full((LANES,), -jnp.inf, jnp.float32))
        mv = _zeros() + jnp.max(acc)
        lo0 = mv - 1.0

        def cp_body(i, cnt):
            c = buf[r, pl.ds(i * LANES, LANES)]
            msk = c > lo0
            plsc.store_compressed(cand.at[pl.ds(cnt, LANES)], c, mask=msk)
            return cnt + jnp.sum(jnp.where(msk, 1, 0))

        cnt = lax.fori_loop(0, CHUNKS, cp_body, jnp.int32(0))
        cand[pl.ds(cnt, LANES)] = lo0
        nch = lax.shift_right_logical(cnt + (LANES - 1), 4)

        def bis_body(j, carry):
            lo, hi = carry
            t = (lo + hi) * 0.5

            def ps(i, a):
                return a + jnp.maximum(cand[pl.ds(i * LANES, LANES)] - t, 0.0)

            sv = _zeros() + jnp.sum(lax.fori_loop(0, nch, ps, _zeros()))
            big = sv >= 1.0
            return jnp.where(big, t, lo), jnp.where(big, hi, t)

        lo, hi = lax.fori_loop(0, BISECT, bis_body, (lo0, mv))

        def ex_body(i, carry):
            kv, sv = carry
            c = cand[pl.ds(i * LANES, LANES)]
            msk = c > hi
            return (kv + jnp.where(msk, 1.0, 0.0),
                    sv + jnp.where(msk, c, 0.0))

        kv, sv = lax.fori_loop(0, nch, ex_body, (_zeros(), _zeros()))
        ks = jnp.maximum(_zeros() + jnp.sum(kv), 1.0)
        ss = _zeros() + jnp.sum(sv)
        t_ex = (ss - 1.0) / ks

        def op_body(i, _):
            sl = pl.ds(i * LANES, LANES)
            buf[r, sl] = jnp.maximum(buf[r, sl] - t_ex, 0.0)
            return 0

        lax.fori_loop(0, CHUNKS, op_body, 0)

    pltpu.sync_copy(buf, out_hbm.at[pl.ds(base, RPW)])


def kernel(logits):
    f = pl.kernel(
        _sparsemax_body,
        out_type=jax.ShapeDtypeStruct((OBS, DIMS), jnp.float32),
        mesh=plsc.VectorSubcoreMesh(core_axis_name="c", subcore_axis_name="s"),
        scratch_types=[
            pltpu.VMEM((RPW, DIMS), jnp.float32),
            pltpu.VMEM((DIMS + LANES,), jnp.float32),
        ],
        compiler_params=pltpu.CompilerParams(needs_layout_passes=False),
    )
    return f(logits)


# speedup vs baseline: 11.7512x; 1.8050x over previous
import jax
import jax.numpy as jnp
from jax import lax
from jax.experimental import pallas as pl
from jax.experimental.pallas import tpu as pltpu
from jax.experimental.pallas import tpu_sc as plsc

OBS = 128
DIMS = 8192
LANES = 16
CHUNKS = DIMS // LANES
NC = 2
NS = 16
NW = NC * NS
RPW = OBS // NW
BISECT = 30
UNROLL = 8


def _zeros():
    return jnp.zeros((LANES,), jnp.float32)


def _sparsemax_body(x_hbm, out_hbm, buf, cand):
    wid = lax.axis_index("s") * NC + lax.axis_index("c")
    base = wid * RPW
    pltpu.sync_copy(x_hbm.at[pl.ds(base, RPW)], buf)

    for r in range(RPW):
        def mx_body(i, acc):
            cs = [buf[r, pl.ds((i * UNROLL + k) * LANES, LANES)]
                  for k in range(UNROLL)]
            while len(cs) > 1:
                cs = [jnp.maximum(cs[j], cs[j + 1]) for j in range(0, len(cs), 2)]
            return jnp.maximum(acc, cs[0])

        acc = lax.fori_loop(0, CHUNKS // UNROLL, mx_body,
                            jnp.full((LANES,), -jnp.inf, jnp.float32))
        mv = _zeros() + jnp.max(acc)
        lo0 = mv - 1.0

        def cp_body(i, cnt):
            cs = [buf[r, pl.ds((i * UNROLL + k) * LANES, LANES)]
                  for k in range(UNROLL)]
            msks = [c > lo0 for c in cs]
            pcs = [plsc.all_reduce_population_count(m)[0] for m in msks]
            off = cnt
            for k in range(UNROLL):
                plsc.store_compressed(cand.at[pl.ds(off, LANES)], cs[k],
                                      mask=msks[k])
                off = off + pcs[k]
            return off

        cnt = lax.fori_loop(0, CHUNKS // UNROLL, cp_body, jnp.int32(0))
        cand[pl.ds(cnt, LANES)] = lo0
        nch = lax.shift_right_logical(cnt + (LANES - 1), 4)

        def bis_body(j, carry):
            lo, hi = carry
            t = (lo + hi) * 0.5

            def ps(i, a):
                return a + jnp.maximum(cand[pl.ds(i * LANES, LANES)] - t, 0.0)

            sv = _zeros() + jnp.sum(lax.fori_loop(0, nch, ps, _zeros()))
            big = sv >= 1.0
            return jnp.where(big, t, lo), jnp.where(big, hi, t)

        lo, hi = lax.fori_loop(0, BISECT, bis_body, (lo0, mv))

        def ex_body(i, carry):
            kv, sv = carry
            c = cand[pl.ds(i * LANES, LANES)]
            msk = c > hi
            return (kv + jnp.where(msk, 1.0, 0.0),
                    sv + jnp.where(msk, c, 0.0))

        kv, sv = lax.fori_loop(0, nch, ex_body, (_zeros(), _zeros()))
        ks = jnp.maximum(_zeros() + jnp.sum(kv), 1.0)
        ss = _zeros() + jnp.sum(sv)
        t_ex = (ss - 1.0) / ks

        def op_body(i, _):
            for k in range(UNROLL):
                sl = pl.ds((i * UNROLL + k) * LANES, LANES)
                buf[r, sl] = jnp.maximum(buf[r, sl] - t_ex, 0.0)
            return 0

        lax.fori_loop(0, CHUNKS // UNROLL, op_body, 0)

    pltpu.sync_copy(buf, out_hbm.at[pl.ds(base, RPW)])


def kernel(logits):
    f = pl.kernel(
        _sparsemax_body,
        out_type=jax.ShapeDtypeStruct((OBS, DIMS), jnp.float32),
        mesh=plsc.VectorSubcoreMesh(core_axis_name="c", subcore_axis_name="s"),
        scratch_types=[
            pltpu.VMEM((RPW, DIMS), jnp.float32),
            pltpu.VMEM((DIMS + LANES,), jnp.float32),
        ],
        compiler_params=pltpu.CompilerParams(needs_layout_passes=False),
    )
    return f(logits)
